# baseline (speedup 1.0000x reference)
import functools

import jax
import jax.numpy as jnp
from jax import lax
from jax.experimental import pallas as pl
from jax.experimental.pallas import tpu as pltpu
from jax.experimental.pallas import tpu_sc as plsc

_BATCH = 4096
_HIST = 200
_DIM = 32
_VOCAB = 1000000
_NC = 2
_NS = 16
_NW = _NC * _NS
_BPW = _BATCH // _NW
_HT = _HIST // 8
_PAIRS = _HIST // 2


def _build():
    mesh = plsc.VectorSubcoreMesh(core_axis_name="c", subcore_axis_name="s")

    @functools.partial(
        pl.kernel,
        out_type=jax.ShapeDtypeStruct((_HIST, _DIM // 8, _NW, 8, _BPW),
                                      jnp.float32),
        mesh=mesh,
        compiler_params=pltpu.CompilerParams(use_tc_tiling_on_sc=False,
                                             needs_layout_passes=False),
        scratch_types=[
            pltpu.VMEM((_HT, 8, _BPW), jnp.int32),
            pltpu.VMEM((_BPW, _DIM), jnp.float32),
            pltpu.VMEM((_BPW, _DIM), jnp.float32),
            pltpu.VMEM((_DIM // 8, 8, _BPW), jnp.float32),
            pltpu.VMEM((_DIM // 8, 8, _BPW), jnp.float32),
            pltpu.SemaphoreType.DMA,
            pltpu.SemaphoreType.DMA,
            pltpu.SemaphoreType.DMA,
            pltpu.SemaphoreType.DMA,
        ],
    )
    def gather_kernel(xs_hbm, table_hbm, out_hbm, idx_v, buf0, buf1,
                      tbuf0, tbuf1, sem_g0, sem_g1, sem_w0, sem_w1):
        wid = lax.axis_index("s") * _NC + lax.axis_index("c")
        bufs = (buf0, buf1)
        tbufs = (tbuf0, tbuf1)
        sems_g = (sem_g0, sem_g1)
        sems_w = (sem_w0, sem_w1)

        for ht in range(_HT):
            pltpu.sync_copy(xs_hbm.at[ht, wid], idx_v.at[ht])

        base_iota = lax.iota(jnp.int32, 16)
        c_idx = [base_iota + cc * 16 for cc in range(8)]

        def fire_gather(h, s):
            return pltpu.async_copy(
                table_hbm.at[idx_v.at[h // 8, h % 8]], bufs[s], sems_g[s])

        def transpose(s):
            src, dst = bufs[s], tbufs[s]
            for tr in range(_DIM // 8):
                for rr in range(8):
                    d_vec = jnp.full((16,), tr * 8 + rr, jnp.int32)
                    for cc in range(8):
                        v = plsc.load_gather(src, [c_idx[cc], d_vec])
                        dst[tr, rr, pl.ds(cc * 16, 16)] = v

        def fire_wb(h, s):
            return pltpu.async_copy(tbufs[s], out_hbm.at[h, :, wid],
                                    sems_w[s])

        def wait_wb(s):
            pltpu.make_async_copy(out_hbm.at[0, :, 0], tbufs[s],
                                  sems_w[s]).wait()

        def pair_body(p, carry):
            h0 = 2 * p
            h1 = h0 + 1

            @pl.when(p > 0)
            def _():
                wait_wb(0)
                wait_wb(1)

            g0 = fire_gather(h0, 0)
            g1 = fire_gather(h1, 1)
            g0.wait()
            transpose(0)
            fire_wb(h0, 0)
            g1.wait()
            transpose(1)
            fire_wb(h1, 1)
            return carry

        lax.fori_loop(0, _PAIRS, pair_body, 0)
        wait_wb(0)
        wait_wb(1)

    return gather_kernel


_gather = _build()


def kernel(x, table):
    xs = (x.astype(jnp.int32).T
          .reshape(_HT, 8, _NW, _BPW)
          .transpose(0, 2, 1, 3))
    t128 = lax.optimization_barrier(table.reshape(_VOCAB // 4, 128))
    tab_lin = t128.reshape(_VOCAB, _DIM)
    out6 = _gather(xs, tab_lin)
    return out6.transpose(2, 4, 0, 1, 3).reshape(_BATCH, _HIST, _DIM)


# speedup vs baseline: 1.1742x; 1.1742x over previous
import functools

import jax
import jax.numpy as jnp
from jax import lax
from jax.experimental import pallas as pl
from jax.experimental.pallas import tpu as pltpu
from jax.experimental.pallas import tpu_sc as plsc

_BATCH = 4096
_HIST = 200
_DIM = 32
_VOCAB = 1000000
_NC = 2
_NS = 16
_NW = _NC * _NS
_BPW = _BATCH // _NW
_HT = _HIST // 8
_PAIRS = _HIST // 2


def _build():
    mesh = plsc.VectorSubcoreMesh(core_axis_name="c", subcore_axis_name="s")

    @functools.partial(
        pl.kernel,
        out_type=jax.ShapeDtypeStruct((_HIST, _DIM // 8, _NW, 8, _BPW),
                                      jnp.float32),
        mesh=mesh,
        compiler_params=pltpu.CompilerParams(use_tc_tiling_on_sc=False,
                                             needs_layout_passes=False),
        scratch_types=[
            pltpu.VMEM((_HT, 8, _BPW), jnp.int32),
            pltpu.VMEM((_BPW, _DIM), jnp.float32),
            pltpu.VMEM((_BPW, _DIM), jnp.float32),
            pltpu.VMEM((_DIM // 8, 8, _BPW), jnp.float32),
            pltpu.VMEM((_DIM // 8, 8, _BPW), jnp.float32),
            pltpu.SemaphoreType.DMA,
            pltpu.SemaphoreType.DMA,
            pltpu.SemaphoreType.DMA,
            pltpu.SemaphoreType.DMA,
        ],
    )
    def gather_kernel(xs_hbm, table_hbm, out_hbm, idx_v, buf0, buf1,
                      tbuf0, tbuf1, sem_g0, sem_g1, sem_w0, sem_w1):
        wid = lax.axis_index("s") * _NC + lax.axis_index("c")
        bufs = (buf0, buf1)
        tbufs = (tbuf0, tbuf1)
        sems_g = (sem_g0, sem_g1)
        sems_w = (sem_w0, sem_w1)

        for ht in range(_HT):
            pltpu.sync_copy(xs_hbm.at[ht, wid], idx_v.at[ht])

        base_iota = lax.iota(jnp.int32, 16)
        c_idx = [base_iota + cc * 16 for cc in range(8)]
        d_vec = [jnp.full((16,), d, jnp.int32) for d in range(_DIM)]

        def fire_gather(h, s):
            return pltpu.async_copy(
                table_hbm.at[idx_v.at[h // 8, h % 8]], bufs[s], sems_g[s])

        def transpose(s):
            src, dst = bufs[s], tbufs[s]
            for tr in range(_DIM // 8):
                for rr in range(8):
                    d = tr * 8 + rr
                    vs = [plsc.load_gather(src, [c_idx[cc], d_vec[d]])
                          for cc in range(8)]
                    for cc in range(8):
                        dst[tr, rr, pl.ds(cc * 16, 16)] = vs[cc]

        def fire_wb(h, s):
            return pltpu.async_copy(tbufs[s], out_hbm.at[h, :, wid],
                                    sems_w[s])

        def wait_wb(s):
            pltpu.make_async_copy(out_hbm.at[0, :, 0], tbufs[s],
                                  sems_w[s]).wait()

        def pair_body(p, carry):
            h0 = 2 * p
            h1 = h0 + 1

            @pl.when(p > 0)
            def _():
                wait_wb(0)
                wait_wb(1)

            g0 = fire_gather(h0, 0)
            g1 = fire_gather(h1, 1)
            g0.wait()
            transpose(0)
            fire_wb(h0, 0)
            g1.wait()
            transpose(1)
            fire_wb(h1, 1)
            return carry

        lax.fori_loop(0, _PAIRS, pair_body, 0)
        wait_wb(0)
        wait_wb(1)

    return gather_kernel


_gather = _build()


def kernel(x, table):
    xs = (x.astype(jnp.int32).T
          .reshape(_HT, 8, _NW, _BPW)
          .transpose(0, 2, 1, 3))
    t128 = lax.optimization_barrier(table.reshape(_VOCAB // 4, 128))
    tab_lin = t128.reshape(_VOCAB, _DIM)
    out6 = _gather(xs, tab_lin)
    return out6.transpose(2, 4, 0, 1, 3).reshape(_BATCH, _HIST, _DIM)


# speedup vs baseline: 1.4553x; 1.2394x over previous
import functools

import jax
import jax.numpy as jnp
from jax import lax
from jax.experimental import pallas as pl
from jax.experimental.pallas import tpu as pltpu
from jax.experimental.pallas import tpu_sc as plsc

_BATCH = 4096
_HIST = 200
_DIM = 32
_VOCAB = 1000000
_NC = 2
_NS = 16
_NW = _NC * _NS
_BPW = _BATCH // _NW
_HT = _HIST // 8
_PAIRS = _HIST // 2


def _build():
    mesh = plsc.VectorSubcoreMesh(core_axis_name="c", subcore_axis_name="s")

    @functools.partial(
        pl.kernel,
        out_type=jax.ShapeDtypeStruct((_HIST, _DIM // 8, _NW, 8, _BPW),
                                      jnp.float32),
        mesh=mesh,
        compiler_params=pltpu.CompilerParams(use_tc_tiling_on_sc=False,
                                             needs_layout_passes=False),
        scratch_types=[
            pltpu.VMEM((_HT, 8, _BPW), jnp.int32),
            pltpu.VMEM((_BPW, _DIM), jnp.float32),
            pltpu.VMEM((_BPW, _DIM), jnp.float32),
            pltpu.VMEM((_DIM // 8, 8, _BPW), jnp.float32),
            pltpu.VMEM((_DIM // 8, 8, _BPW), jnp.float32),
            pltpu.SemaphoreType.DMA,
            pltpu.SemaphoreType.DMA,
            pltpu.SemaphoreType.DMA,
            pltpu.SemaphoreType.DMA,
        ],
    )
    def gather_kernel(xs_hbm, table_hbm, out_hbm, idx_v, buf0, buf1,
                      tbuf0, tbuf1, sem_g0, sem_g1, sem_w0, sem_w1):
        wid = lax.axis_index("s") * _NC + lax.axis_index("c")
        bufs = (buf0, buf1)
        tbufs = (tbuf0, tbuf1)
        sems_g = (sem_g0, sem_g1)
        sems_w = (sem_w0, sem_w1)

        for ht in range(_HT):
            pltpu.sync_copy(xs_hbm.at[ht, wid], idx_v.at[ht])

        base_iota = lax.iota(jnp.int32, 16)
        c_idx = [base_iota + cc * 16 for cc in range(8)]
        rot = [(base_iota + j) % 16 for j in range(16)]

        def fire_gather(h, s):
            return pltpu.async_copy(
                table_hbm.at[idx_v.at[h // 8, h % 8]], bufs[s], sems_g[s])

        def transpose(s):
            src, dst = bufs[s], tbufs[s]
            for d0 in (0, 16):
                for j in range(16):
                    d = rot[j] + d0
                    tr_vec = d >> 3
                    rr_vec = d & 7
                    vs = [plsc.load_gather(src, [c_idx[cc], d])
                          for cc in range(8)]
                    for cc in range(8):
                        plsc.store_scatter(
                            dst, [tr_vec, rr_vec, c_idx[cc]], vs[cc])

        def fire_wb(h, s):
            return pltpu.async_copy(tbufs[s], out_hbm.at[h, :, wid],
                                    sems_w[s])

        def wait_wb(s):
            pltpu.make_async_copy(out_hbm.at[0, :, 0], tbufs[s],
                                  sems_w[s]).wait()

        def pair_body(p, carry):
            h0 = 2 * p
            h1 = h0 + 1

            @pl.when(p > 0)
            def _():
                wait_wb(0)
                wait_wb(1)

            g0 = fire_gather(h0, 0)
            g1 = fire_gather(h1, 1)
            g0.wait()
            transpose(0)
            fire_wb(h0, 0)
            g1.wait()
            transpose(1)
            fire_wb(h1, 1)
            return carry

        lax.fori_loop(0, _PAIRS, pair_body, 0)
        wait_wb(0)
        wait_wb(1)

    return gather_kernel


_gather = _build()


def kernel(x, table):
    xs = (x.astype(jnp.int32).T
          .reshape(_HT, 8, _NW, _BPW)
          .transpose(0, 2, 1, 3))
    t128 = lax.optimization_barrier(table.reshape(_VOCAB // 4, 128))
    tab_lin = t128.reshape(_VOCAB, _DIM)
    out6 = _gather(xs, tab_lin)
    return out6.transpose(2, 4, 0, 1, 3).reshape(_BATCH, _HIST, _DIM)


# speedup vs baseline: 1.8984x; 1.3045x over previous
import functools

import jax
import jax.numpy as jnp
from jax import lax
from jax.experimental import pallas as pl
from jax.experimental.pallas import tpu as pltpu
from jax.experimental.pallas import tpu_sc as plsc

_BATCH = 4096
_HIST = 200
_DIM = 32
_VOCAB = 1000000
_NC = 2
_NS = 16
_NW = _NC * _NS
_BPW = _BATCH // _NW
_HT = _HIST // 8
_PAIRS = _HIST // 2


def _build():
    mesh = plsc.VectorSubcoreMesh(core_axis_name="c", subcore_axis_name="s")

    @functools.partial(
        pl.kernel,
        out_type=jax.ShapeDtypeStruct((_HIST, _DIM // 8, _NW, 8, _BPW),
                                      jnp.float32),
        mesh=mesh,
        compiler_params=pltpu.CompilerParams(use_tc_tiling_on_sc=False,
                                             needs_layout_passes=False),
        scratch_types=[
            pltpu.VMEM((_HT, 8, _BPW), jnp.int32),
            pltpu.VMEM((_BPW, _DIM), jnp.float32),
            pltpu.VMEM((_BPW, _DIM), jnp.float32),
            pltpu.VMEM((_DIM // 8, 8, _BPW), jnp.float32),
            pltpu.VMEM((_DIM // 8, 8, _BPW), jnp.float32),
            pltpu.SemaphoreType.DMA,
            pltpu.SemaphoreType.DMA,
            pltpu.SemaphoreType.DMA,
            pltpu.SemaphoreType.DMA,
        ],
    )
    def gather_kernel(xs_hbm, table_hbm, out_hbm, idx_v, buf0, buf1,
                      tbuf0, tbuf1, sem_g0, sem_g1, sem_w0, sem_w1):
        wid = lax.axis_index("s") * _NC + lax.axis_index("c")
        bufs = (buf0, buf1)
        tbufs = (tbuf0, tbuf1)
        sems_g = (sem_g0, sem_g1)
        sems_w = (sem_w0, sem_w1)

        for ht in range(_HT):
            pltpu.sync_copy(xs_hbm.at[ht, wid], idx_v.at[ht])

        base_iota = lax.iota(jnp.int32, 16)
        c_idx = [base_iota + cc * 16 for cc in range(8)]
        rot = [(base_iota + j) % 16 for j in range(16)]

        def fire_gather(h, s):
            return pltpu.async_copy(
                table_hbm.at[idx_v.at[h // 8, h % 8]], bufs[s], sems_g[s])

        def transpose(s):
            src, dst = bufs[s], tbufs[s]
            for d0 in (0, 16):

                @plsc.parallel_loop(0, 16, unroll=4)
                def _(j):
                    d = ((base_iota + j) & 15) + d0
                    tr_vec = d >> 3
                    rr_vec = d & 7
                    vs = [plsc.load_gather(src, [c_idx[cc], d])
                          for cc in range(8)]
                    for cc in range(8):
                        plsc.store_scatter(
                            dst, [tr_vec, rr_vec, c_idx[cc]], vs[cc])

        def fire_wb(h, s):
            return pltpu.async_copy(tbufs[s], out_hbm.at[h, :, wid],
                                    sems_w[s])

        def wait_wb(s):
            pltpu.make_async_copy(out_hbm.at[0, :, 0], tbufs[s],
                                  sems_w[s]).wait()

        def pair_body(p, carry):
            h0 = 2 * p
            h1 = h0 + 1

            @pl.when(p > 0)
            def _():
                wait_wb(0)
                wait_wb(1)

            g0 = fire_gather(h0, 0)
            g1 = fire_gather(h1, 1)
            g0.wait()
            transpose(0)
            fire_wb(h0, 0)
            g1.wait()
            transpose(1)
            fire_wb(h1, 1)
            return carry

        lax.fori_loop(0, _PAIRS, pair_body, 0)
        wait_wb(0)
        wait_wb(1)

    return gather_kernel


_gather = _build()


def kernel(x, table):
    xs = (x.astype(jnp.int32).T
          .reshape(_HT, 8, _NW, _BPW)
          .transpose(0, 2, 1, 3))
    t128 = lax.optimization_barrier(table.reshape(_VOCAB // 4, 128))
    tab_lin = t128.reshape(_VOCAB, _DIM)
    out6 = _gather(xs, tab_lin)
    return out6.transpose(2, 4, 0, 1, 3).reshape(_BATCH, _HIST, _DIM)
